# baseline (speedup 1.0000x reference)
import jax
import jax.numpy as jnp
from jax.experimental import pallas as pl
from jax.experimental.pallas import tpu as pltpu

_B, _S, _D, _M, _P = 4096, 4, 512, 5000, 336
_TEMP = 0.07
_GATE_SHARP = 10.0
_BB = 256
_SC = 2


def _make_body(scales, has_prev):
    def body(*refs):
        thr_ref, q_ref, ka_ref, kb_ref, vta_ref, vtb_ref = refs[:6]
        o_ref = refs[7] if has_prev else refs[6]
        kinva, kinvb, vbta, vbtb = refs[-4:]
        i = pl.program_id(0)
        k_refs = [ka_ref, kb_ref]
        vt_refs = [vta_ref, vtb_ref]
        kinvs = [kinva, kinvb]
        vbts = [vbta, vbtb]

        @pl.when(i == 0)
        def _init():
            for j in range(_SC):
                k = k_refs[j][...]
                s2 = jax.lax.dot_general(
                    jnp.ones((8, _D), jnp.float32), k * k,
                    (((1,), (1,)), ((), ())),
                    preferred_element_type=jnp.float32)
                kinvs[j][...] = 1.0 / (jnp.sqrt(s2[:1]) + 1e-8)
                vbts[j][:_P, :] = vt_refs[j][...].astype(jnp.bfloat16)
                vbts[j][_P:, :] = jnp.ones((1, _M), jnp.bfloat16)

        pieces = []
        for j, s in enumerate(scales):
            q = q_ref[:, s, :]
            qs = (1.0 / _TEMP) / (
                jnp.sqrt(jnp.sum(q * q, axis=1, keepdims=True)) + 1e-8)
            raw = jax.lax.dot_general(
                q, k_refs[j][...], (((1,), (1,)), ((), ())),
                preferred_element_type=jnp.float32)
            x = raw * qs * kinvs[j][...]
            e = jnp.exp(x)
            gate = jax.nn.sigmoid(
                x * (_GATE_SHARP * _TEMP) - thr_ref[s] * _GATE_SHARP)
            eg = e * gate
            z = jnp.sum(e, axis=1, keepdims=True)
            numa = jax.lax.dot_general(
                eg.astype(jnp.bfloat16), vbts[j][...],
                (((1,), (1,)), ((), ())),
                preferred_element_type=jnp.float32)
            pieces.append(numa[:, :_P] / (numa[:, _P:] + 1e-8 * z))
        o_ref[...] = jnp.concatenate(
            pieces, axis=1).reshape(_BB, 1, _SC, _P)

    return body


@jax.jit
def kernel(query, thresholds, keys_0, keys_1, keys_2, keys_3,
           values_0, values_1, values_2, values_3):
    keys = [keys_0, keys_1, keys_2, keys_3]
    values = [values_0, values_1, values_2, values_3]
    const = lambda i: (0, 0)

    out = None
    for h, scales in enumerate(((0, 1), (2, 3))):
        in_specs = [
            pl.BlockSpec(memory_space=pltpu.SMEM),
            pl.BlockSpec((_BB, _S, _D), lambda i: (i, 0, 0)),
            pl.BlockSpec((_M, _D), const),
            pl.BlockSpec((_M, _D), const),
            pl.BlockSpec((_P, _M), const),
            pl.BlockSpec((_P, _M), const),
        ]
        args = [thresholds, query, keys[scales[0]], keys[scales[1]],
                values[scales[0]].T, values[scales[1]].T]
        io_aliases = {}
        if out is not None:
            in_specs.append(pl.BlockSpec(memory_space=pl.ANY))
            args.append(out)
            io_aliases = {6: 0}
        out = pl.pallas_call(
            _make_body(scales, out is not None),
            grid=(_B // _BB,),
            in_specs=in_specs,
            out_specs=pl.BlockSpec((_BB, 1, _SC, _P),
                                   lambda i, h=h: (i, h, 0, 0)),
            out_shape=jax.ShapeDtypeStruct((_B, 2, _SC, _P), jnp.float32),
            scratch_shapes=(
                [pltpu.VMEM((1, _M), jnp.float32) for _ in range(_SC)]
                + [pltpu.VMEM((_P + 1, _M), jnp.bfloat16) for _ in range(_SC)]
            ),
            input_output_aliases=io_aliases,
        )(*args)

    return out.reshape(_B, _S, _P)


# speedup vs baseline: 1.0317x; 1.0317x over previous
import jax
import jax.numpy as jnp
from jax.experimental import pallas as pl
from jax.experimental.pallas import tpu as pltpu

_B, _S, _D, _M, _P = 4096, 4, 512, 5000, 336
_TEMP = 0.07
_GATE_SHARP = 10.0
_BB = 256
_SC = 2


def _make_body(scales, has_prev):
    def body(*refs):
        thr_ref, q_ref, ka_ref, kb_ref, vta_ref, vtb_ref = refs[:6]
        o_ref = refs[7] if has_prev else refs[6]
        kinva, kinvb, vbta, vbtb = refs[-4:]
        i = pl.program_id(0)
        k_refs = [ka_ref, kb_ref]
        vt_refs = [vta_ref, vtb_ref]
        kinvs = [kinva, kinvb]
        vbts = [vbta, vbtb]

        @pl.when(i == 0)
        def _init():
            for j in range(_SC):
                k = k_refs[j][...]
                s2 = jax.lax.dot_general(
                    jnp.ones((8, _D), jnp.float32), k * k,
                    (((1,), (1,)), ((), ())),
                    preferred_element_type=jnp.float32)
                kinvs[j][...] = 1.0 / (jnp.sqrt(s2[:1]) + 1e-8)
                vbts[j][:_P, :] = vt_refs[j][...].astype(jnp.bfloat16)
                vbts[j][_P:, :] = jnp.ones((1, _M), jnp.bfloat16)

        pieces = []
        for j, s in enumerate(scales):
            q = q_ref[:, s, :]
            qs = (1.0 / _TEMP) / (
                jnp.sqrt(jnp.sum(q * q, axis=1, keepdims=True)) + 1e-8)
            raw = jax.lax.dot_general(
                q, k_refs[j][...], (((1,), (1,)), ((), ())),
                preferred_element_type=jnp.float32)
            x = raw * qs * kinvs[j][...]
            e = jnp.exp(x)
            gate = jax.nn.sigmoid(
                x * (_GATE_SHARP * _TEMP) - thr_ref[s] * _GATE_SHARP)
            eg = e * gate
            z = jnp.sum(e, axis=1, keepdims=True)
            numa = jax.lax.dot_general(
                eg.astype(jnp.bfloat16), vbts[j][...],
                (((1,), (1,)), ((), ())),
                preferred_element_type=jnp.float32)
            pieces.append(numa[:, :_P] / (numa[:, _P:] + 1e-8 * z))
        for j in range(_SC):
            o_ref[:, 0, j, :] = pieces[j]

    return body


@jax.jit
def kernel(query, thresholds, keys_0, keys_1, keys_2, keys_3,
           values_0, values_1, values_2, values_3):
    keys = [keys_0, keys_1, keys_2, keys_3]
    values = [values_0, values_1, values_2, values_3]
    const = lambda i: (0, 0)

    out = None
    for h, scales in enumerate(((0, 1), (2, 3))):
        in_specs = [
            pl.BlockSpec(memory_space=pltpu.SMEM),
            pl.BlockSpec((_BB, _S, _D), lambda i: (i, 0, 0)),
            pl.BlockSpec((_M, _D), const),
            pl.BlockSpec((_M, _D), const),
            pl.BlockSpec((_P, _M), const),
            pl.BlockSpec((_P, _M), const),
        ]
        args = [thresholds, query, keys[scales[0]], keys[scales[1]],
                values[scales[0]].T, values[scales[1]].T]
        io_aliases = {}
        if out is not None:
            in_specs.append(pl.BlockSpec(memory_space=pl.ANY))
            args.append(out)
            io_aliases = {6: 0}
        out = pl.pallas_call(
            _make_body(scales, out is not None),
            grid=(_B // _BB,),
            in_specs=in_specs,
            out_specs=pl.BlockSpec((_BB, 1, _SC, _P),
                                   lambda i, h=h: (i, h, 0, 0)),
            out_shape=jax.ShapeDtypeStruct((_B, 2, _SC, _P), jnp.float32),
            scratch_shapes=(
                [pltpu.VMEM((1, _M), jnp.float32) for _ in range(_SC)]
                + [pltpu.VMEM((_P + 1, _M), jnp.bfloat16) for _ in range(_SC)]
            ),
            input_output_aliases=io_aliases,
        )(*args)

    return out.reshape(_B, _S, _P)


# speedup vs baseline: 1.0353x; 1.0034x over previous
import jax
import jax.numpy as jnp
from jax.experimental import pallas as pl
from jax.experimental.pallas import tpu as pltpu

_B, _S, _D, _M, _P = 4096, 4, 512, 5000, 336
_TEMP = 0.07
_GATE_SHARP = 10.0
_BB = 256
_SC = 2


def _make_body(scales, has_prev):
    def body(*refs):
        thr_ref, q_ref, ka_ref, kb_ref, vta_ref, vtb_ref = refs[:6]
        o_ref = refs[7] if has_prev else refs[6]
        kinva, kinvb, vbta, vbtb = refs[-4:]
        i = pl.program_id(0)
        k_refs = [ka_ref, kb_ref]
        vt_refs = [vta_ref, vtb_ref]
        kinvs = [kinva, kinvb]
        vbts = [vbta, vbtb]

        @pl.when(i == 0)
        def _init():
            for j in range(_SC):
                k = k_refs[j][...]
                s2 = jax.lax.dot_general(
                    jnp.ones((8, _D), jnp.float32), k * k,
                    (((1,), (1,)), ((), ())),
                    preferred_element_type=jnp.float32)
                kinvs[j][...] = 1.0 / (jnp.sqrt(s2[:1]) + 1e-8)
                vbts[j][:_P, :] = vt_refs[j][...].astype(jnp.bfloat16)
                vbts[j][_P:, :] = jnp.ones((1, _M), jnp.bfloat16)

        pieces = []
        for j, s in enumerate(scales):
            q = q_ref[:, s, :]
            qs = (1.0 / _TEMP) / (
                jnp.sqrt(jnp.sum(q * q, axis=1, keepdims=True)) + 1e-8)
            raw = jax.lax.dot_general(
                q, k_refs[j][...], (((1,), (1,)), ((), ())),
                precision=jax.lax.Precision.DEFAULT,
                preferred_element_type=jnp.float32)
            x = raw * qs * kinvs[j][...]
            e = jnp.exp(x)
            gate = jax.nn.sigmoid(
                x * (_GATE_SHARP * _TEMP) - thr_ref[s] * _GATE_SHARP)
            eg = e * gate
            z = jnp.sum(e, axis=1, keepdims=True)
            numa = jax.lax.dot_general(
                eg.astype(jnp.bfloat16), vbts[j][...],
                (((1,), (1,)), ((), ())),
                preferred_element_type=jnp.float32)
            pieces.append(numa[:, :_P] / (numa[:, _P:] + 1e-8 * z))
        for j in range(_SC):
            o_ref[:, 0, j, :] = pieces[j]

    return body


@jax.jit
def kernel(query, thresholds, keys_0, keys_1, keys_2, keys_3,
           values_0, values_1, values_2, values_3):
    keys = [keys_0, keys_1, keys_2, keys_3]
    values = [values_0, values_1, values_2, values_3]
    const = lambda i: (0, 0)

    out = None
    for h, scales in enumerate(((0, 1), (2, 3))):
        in_specs = [
            pl.BlockSpec(memory_space=pltpu.SMEM),
            pl.BlockSpec((_BB, _S, _D), lambda i: (i, 0, 0)),
            pl.BlockSpec((_M, _D), const),
            pl.BlockSpec((_M, _D), const),
            pl.BlockSpec((_P, _M), const),
            pl.BlockSpec((_P, _M), const),
        ]
        args = [thresholds, query, keys[scales[0]], keys[scales[1]],
                values[scales[0]].T, values[scales[1]].T]
        io_aliases = {}
        if out is not None:
            in_specs.append(pl.BlockSpec(memory_space=pl.ANY))
            args.append(out)
            io_aliases = {6: 0}
        out = pl.pallas_call(
            _make_body(scales, out is not None),
            grid=(_B // _BB,),
            in_specs=in_specs,
            out_specs=pl.BlockSpec((_BB, 1, _SC, _P),
                                   lambda i, h=h: (i, h, 0, 0)),
            out_shape=jax.ShapeDtypeStruct((_B, 2, _SC, _P), jnp.float32),
            scratch_shapes=(
                [pltpu.VMEM((1, _M), jnp.float32) for _ in range(_SC)]
                + [pltpu.VMEM((_P + 1, _M), jnp.bfloat16) for _ in range(_SC)]
            ),
            input_output_aliases=io_aliases,
        )(*args)

    return out.reshape(_B, _S, _P)
